# baseline (speedup 1.0000x reference)
import jax
import jax.numpy as jnp
from jax.experimental import pallas as pl
from jax.experimental.pallas import tpu as pltpu
from jax.experimental.pallas import tpu_sc as plsc

D_MODEL = 64
SCALE = 8.0
WINDOW = 128
TILE_C = 2048


def _scale_widen(table_t):
    V = table_t.shape[1]

    def body(in_ref, out_ref):
        t = jnp.transpose(in_ref[...]) * SCALE
        out_ref[...] = jnp.concatenate([t, t], axis=1)

    grid = (V + TILE_C - 1) // TILE_C
    return pl.pallas_call(
        body,
        grid=(grid,),
        in_specs=[pl.BlockSpec((D_MODEL, TILE_C), lambda i: (0, i))],
        out_specs=pl.BlockSpec((TILE_C, 128), lambda i: (i, 0)),
        out_shape=jax.ShapeDtypeStruct((V, 128), jnp.float32),
        compiler_params=pltpu.CompilerParams(
            dimension_semantics=("parallel",)),
    )(table_t)


def kernel(x, table):
    B, S = x.shape
    N = B * S
    idx = x.reshape(1, N).astype(jnp.int32)

    wide = _scale_widen(table.T)

    mesh = plsc.VectorSubcoreMesh(core_axis_name="c", subcore_axis_name="s")

    @pl.kernel(
        out_type=jax.ShapeDtypeStruct((N, 128), jnp.float32),
        mesh=mesh,
        compiler_params=pltpu.CompilerParams(use_tc_tiling_on_sc=False),
    )
    def emb_kernel(tbl_hbm, i_hbm, o_hbm):
        def body(i_vmem, o_vmem):
            pltpu.sync_copy(tbl_hbm.at[i_vmem.at[0]], o_vmem)

        pltpu.emit_pipeline(
            body,
            grid=(N // WINDOW,),
            in_specs=[pl.BlockSpec((1, WINDOW), lambda i: (0, i))],
            out_specs=[pl.BlockSpec((WINDOW, 128), lambda i: (i, 0))],
            core_axis_name=("c", "s"),
            dimension_semantics=(pltpu.PARALLEL,),
        )(i_hbm, o_hbm)

    out = emb_kernel(wide, idx)
    return out[:, :D_MODEL].reshape(B, S, D_MODEL)


# speedup vs baseline: 1.0496x; 1.0496x over previous
import jax
import jax.numpy as jnp
from jax.experimental import pallas as pl
from jax.experimental.pallas import tpu as pltpu
from jax.experimental.pallas import tpu_sc as plsc

D_MODEL = 64
SCALE = 8.0
WINDOW = 128
TILE_C = 2048


def _scale_widen(table_t):
    V = table_t.shape[1]

    def body(in_ref, out_ref):
        t = jnp.transpose(in_ref[...]) * SCALE
        out_ref[:, 0:D_MODEL] = t

    grid = (V + TILE_C - 1) // TILE_C
    return pl.pallas_call(
        body,
        grid=(grid,),
        in_specs=[pl.BlockSpec((D_MODEL, TILE_C), lambda i: (0, i))],
        out_specs=pl.BlockSpec((TILE_C, 128), lambda i: (i, 0)),
        out_shape=jax.ShapeDtypeStruct((V, 128), jnp.float32),
        compiler_params=pltpu.CompilerParams(
            dimension_semantics=("parallel",)),
    )(table_t)


def kernel(x, table):
    B, S = x.shape
    N = B * S
    idx = x.reshape(1, N).astype(jnp.int32)

    wide = _scale_widen(table.T)

    mesh = plsc.VectorSubcoreMesh(core_axis_name="c", subcore_axis_name="s")

    @pl.kernel(
        out_type=jax.ShapeDtypeStruct((N, 128), jnp.float32),
        mesh=mesh,
        compiler_params=pltpu.CompilerParams(use_tc_tiling_on_sc=False),
    )
    def emb_kernel(tbl_hbm, i_hbm, o_hbm):
        def body(i_vmem, o_vmem):
            pltpu.sync_copy(tbl_hbm.at[i_vmem.at[0]], o_vmem)

        pltpu.emit_pipeline(
            body,
            grid=(N // WINDOW,),
            in_specs=[pl.BlockSpec((1, WINDOW), lambda i: (0, i))],
            out_specs=[pl.BlockSpec((WINDOW, 128), lambda i: (i, 0))],
            core_axis_name=("c", "s"),
            dimension_semantics=(pltpu.PARALLEL,),
        )(i_hbm, o_hbm)

    out = emb_kernel(wide, idx)
    return out[:, :D_MODEL].reshape(B, S, D_MODEL)


# speedup vs baseline: 1.2973x; 1.2359x over previous
import jax
import jax.numpy as jnp
from jax.experimental import pallas as pl
from jax.experimental.pallas import tpu as pltpu
from jax.experimental.pallas import tpu_sc as plsc

D_MODEL = 64
SCALE = 8.0
WINDOW = 128
TILE_C = 8192


def _scale_widen(table_t):
    V = table_t.shape[1]

    def body(in_ref, out_ref):
        t = jnp.transpose(in_ref[...]) * SCALE
        out_ref[:, 0:D_MODEL] = t

    grid = (V + TILE_C - 1) // TILE_C
    return pl.pallas_call(
        body,
        grid=(grid,),
        in_specs=[pl.BlockSpec((D_MODEL, TILE_C), lambda i: (0, i))],
        out_specs=pl.BlockSpec((TILE_C, 128), lambda i: (i, 0)),
        out_shape=jax.ShapeDtypeStruct((V, 128), jnp.float32),
        compiler_params=pltpu.CompilerParams(
            dimension_semantics=("parallel",)),
    )(table_t)


def kernel(x, table):
    B, S = x.shape
    N = B * S
    idx = x.reshape(1, N).astype(jnp.int32)

    wide = _scale_widen(table.T)

    mesh = plsc.VectorSubcoreMesh(core_axis_name="c", subcore_axis_name="s")

    @pl.kernel(
        out_type=jax.ShapeDtypeStruct((N, 128), jnp.float32),
        mesh=mesh,
        compiler_params=pltpu.CompilerParams(use_tc_tiling_on_sc=False),
    )
    def emb_kernel(tbl_hbm, i_hbm, o_hbm):
        def body(i_vmem, o_vmem):
            pltpu.sync_copy(tbl_hbm.at[i_vmem.at[0]], o_vmem)

        pltpu.emit_pipeline(
            body,
            grid=(N // WINDOW,),
            in_specs=[pl.BlockSpec((1, WINDOW), lambda i: (0, i))],
            out_specs=[pl.BlockSpec((WINDOW, 128), lambda i: (i, 0))],
            core_axis_name=("c", "s"),
            dimension_semantics=(pltpu.PARALLEL,),
        )(i_hbm, o_hbm)

    out = emb_kernel(wide, idx)
    return out[:, :D_MODEL].reshape(B, S, D_MODEL)


# speedup vs baseline: 1.3267x; 1.0226x over previous
import jax
import jax.numpy as jnp
from jax.experimental import pallas as pl
from jax.experimental.pallas import tpu as pltpu
from jax.experimental.pallas import tpu_sc as plsc

D_MODEL = 64
SCALE = 8.0
WINDOW = 128
TILE_C = 16384


def _scale_widen(table_t):
    V = table_t.shape[1]

    def body(in_ref, out_ref):
        t = jnp.transpose(in_ref[...]) * SCALE
        out_ref[:, 0:D_MODEL] = t

    grid = (V + TILE_C - 1) // TILE_C
    return pl.pallas_call(
        body,
        grid=(grid,),
        in_specs=[pl.BlockSpec((D_MODEL, TILE_C), lambda i: (0, i))],
        out_specs=pl.BlockSpec((TILE_C, 128), lambda i: (i, 0)),
        out_shape=jax.ShapeDtypeStruct((V, 128), jnp.float32),
        compiler_params=pltpu.CompilerParams(
            dimension_semantics=("parallel",)),
    )(table_t)


def kernel(x, table):
    B, S = x.shape
    N = B * S
    idx = x.reshape(1, N).astype(jnp.int32)

    wide = _scale_widen(table.T)

    mesh = plsc.VectorSubcoreMesh(core_axis_name="c", subcore_axis_name="s")

    @pl.kernel(
        out_type=jax.ShapeDtypeStruct((N, 128), jnp.float32),
        mesh=mesh,
        compiler_params=pltpu.CompilerParams(use_tc_tiling_on_sc=False),
    )
    def emb_kernel(tbl_hbm, i_hbm, o_hbm):
        def body(i_vmem, o_vmem):
            pltpu.sync_copy(tbl_hbm.at[i_vmem.at[0]], o_vmem)

        pltpu.emit_pipeline(
            body,
            grid=(N // WINDOW,),
            in_specs=[pl.BlockSpec((1, WINDOW), lambda i: (0, i))],
            out_specs=[pl.BlockSpec((WINDOW, 128), lambda i: (i, 0))],
            core_axis_name=("c", "s"),
            dimension_semantics=(pltpu.PARALLEL,),
        )(i_hbm, o_hbm)

    out = emb_kernel(wide, idx)
    return out[:, :D_MODEL].reshape(B, S, D_MODEL)


# speedup vs baseline: 1.3396x; 1.0098x over previous
import jax
import jax.numpy as jnp
from jax.experimental import pallas as pl
from jax.experimental.pallas import tpu as pltpu
from jax.experimental.pallas import tpu_sc as plsc

D_MODEL = 64
SCALE = 8.0
WINDOW = 128
TILE_C = 32768


def _scale_widen(table_t):
    V = table_t.shape[1]

    def body(in_ref, out_ref):
        t = jnp.transpose(in_ref[...]) * SCALE
        out_ref[:, 0:D_MODEL] = t

    grid = (V + TILE_C - 1) // TILE_C
    return pl.pallas_call(
        body,
        grid=(grid,),
        in_specs=[pl.BlockSpec((D_MODEL, TILE_C), lambda i: (0, i))],
        out_specs=pl.BlockSpec((TILE_C, 128), lambda i: (i, 0)),
        out_shape=jax.ShapeDtypeStruct((V, 128), jnp.float32),
        compiler_params=pltpu.CompilerParams(
            dimension_semantics=("parallel",)),
    )(table_t)


def kernel(x, table):
    B, S = x.shape
    N = B * S
    idx = x.reshape(1, N).astype(jnp.int32)

    wide = _scale_widen(table.T)

    mesh = plsc.VectorSubcoreMesh(core_axis_name="c", subcore_axis_name="s")

    @pl.kernel(
        out_type=jax.ShapeDtypeStruct((N, 128), jnp.float32),
        mesh=mesh,
        compiler_params=pltpu.CompilerParams(use_tc_tiling_on_sc=False),
    )
    def emb_kernel(tbl_hbm, i_hbm, o_hbm):
        def body(i_vmem, o_vmem):
            pltpu.sync_copy(tbl_hbm.at[i_vmem.at[0]], o_vmem)

        pltpu.emit_pipeline(
            body,
            grid=(N // WINDOW,),
            in_specs=[pl.BlockSpec((1, WINDOW), lambda i: (0, i))],
            out_specs=[pl.BlockSpec((WINDOW, 128), lambda i: (i, 0))],
            core_axis_name=("c", "s"),
            dimension_semantics=(pltpu.PARALLEL,),
        )(i_hbm, o_hbm)

    out = emb_kernel(wide, idx)
    return out[:, :D_MODEL].reshape(B, S, D_MODEL)


# speedup vs baseline: 1.4868x; 1.1098x over previous
import jax
import jax.numpy as jnp
from jax import lax
from jax.experimental import pallas as pl
from jax.experimental.pallas import tpu as pltpu
from jax.experimental.pallas import tpu_sc as plsc

D_MODEL = 64
SCALE = 8.0
WINDOW = 128
TILE_C = 32768
NBUF = 4
LOOKAHEAD = 2
NWORKERS = 32


def _scale_widen(table_t):
    V = table_t.shape[1]

    def body(in_ref, out_ref):
        t = jnp.transpose(in_ref[...]) * SCALE
        out_ref[:, 0:D_MODEL] = t

    grid = (V + TILE_C - 1) // TILE_C
    return pl.pallas_call(
        body,
        grid=(grid,),
        in_specs=[pl.BlockSpec((D_MODEL, TILE_C), lambda i: (0, i))],
        out_specs=pl.BlockSpec((TILE_C, 128), lambda i: (i, 0)),
        out_shape=jax.ShapeDtypeStruct((V, 128), jnp.float32),
        compiler_params=pltpu.CompilerParams(
            dimension_semantics=("parallel",)),
    )(table_t)


def kernel(x, table):
    B, S = x.shape
    N = B * S
    idx = x.reshape(1, N).astype(jnp.int32)
    per_w = N // NWORKERS
    n_win = per_w // WINDOW

    wide = _scale_widen(table.T)

    mesh = plsc.VectorSubcoreMesh(core_axis_name="c", subcore_axis_name="s")

    @pl.kernel(
        out_type=jax.ShapeDtypeStruct((N, 128), jnp.float32),
        mesh=mesh,
        scratch_types=(
            [pltpu.VMEM((per_w,), jnp.int32)]
            + [pltpu.VMEM((WINDOW, 128), jnp.float32) for _ in range(NBUF)]
            + [pltpu.SemaphoreType.DMA for _ in range(2 * NBUF + 1)]
        ),
        compiler_params=pltpu.CompilerParams(use_tc_tiling_on_sc=False),
    )
    def emb_kernel(tbl_hbm, i_hbm, o_hbm, idx_v, b0, b1, b2, b3,
                   g0, g1, g2, g3, o0, o1, o2, o3, isem):
        bufs = (b0, b1, b2, b3)
        gsems = (g0, g1, g2, g3)
        osems = (o0, o1, o2, o3)
        wid = lax.axis_index("s") * 2 + lax.axis_index("c")
        base = wid * per_w

        pltpu.async_copy(i_hbm.at[0, pl.ds(base, per_w)], idx_v, isem).wait()

        def idx_slice(k):
            return idx_v.at[pl.ds(k * WINDOW, WINDOW)]

        def out_slice(k):
            return o_hbm.at[pl.ds(base + k * WINDOW, WINDOW)]

        def fire_gather(k, b):
            pltpu.async_copy(tbl_hbm.at[idx_slice(k)], bufs[b], gsems[b])

        for k in range(LOOKAHEAD):
            fire_gather(k, k % NBUF)

        @pl.loop(0, n_win, step=NBUF)
        def _(k0):
            for b in range(NBUF):
                k = k0 + b
                pltpu.make_async_copy(
                    tbl_hbm.at[idx_slice(k)], bufs[b], gsems[b]).wait()
                pltpu.async_copy(bufs[b], out_slice(k), osems[b])

                bn = (b + LOOKAHEAD) % NBUF
                kn = k + LOOKAHEAD

                @pl.when(k >= NBUF - LOOKAHEAD)
                def _():
                    pltpu.make_async_copy(
                        bufs[bn], out_slice(kn - NBUF), osems[bn]).wait()

                @pl.when(kn < n_win)
                def _():
                    fire_gather(kn, bn)

        for k in range(n_win - LOOKAHEAD, n_win):
            b = k % NBUF
            pltpu.make_async_copy(bufs[b], out_slice(k), osems[b]).wait()

    out = emb_kernel(wide, idx)
    return out[:, :D_MODEL].reshape(B, S, D_MODEL)


# speedup vs baseline: 1.8616x; 1.2521x over previous
import jax
import jax.numpy as jnp
from jax import lax
from jax.experimental import pallas as pl
from jax.experimental.pallas import tpu as pltpu
from jax.experimental.pallas import tpu_sc as plsc

D_MODEL = 64
SCALE = 8.0
WINDOW = 128
TILE_C = 32768
NBUF = 8
LOOKAHEAD = 4
NWORKERS = 32


def _scale_widen(table_t):
    V = table_t.shape[1]

    def body(in_ref, out_ref):
        t = jnp.transpose(in_ref[...]) * SCALE
        out_ref[:, 0:D_MODEL] = t

    grid = (V + TILE_C - 1) // TILE_C
    return pl.pallas_call(
        body,
        grid=(grid,),
        in_specs=[pl.BlockSpec((D_MODEL, TILE_C), lambda i: (0, i))],
        out_specs=pl.BlockSpec((TILE_C, 128), lambda i: (i, 0)),
        out_shape=jax.ShapeDtypeStruct((V, 128), jnp.float32),
        compiler_params=pltpu.CompilerParams(
            dimension_semantics=("parallel",)),
    )(table_t)


def kernel(x, table):
    B, S = x.shape
    N = B * S
    V = table.shape[0]
    idx = (x.reshape(1, N).astype(jnp.int32)) * 2
    per_w = N // NWORKERS
    n_win = per_w // WINDOW

    wide = _scale_widen(table.T)
    tbl2 = wide.reshape(2 * V, D_MODEL)

    mesh = plsc.VectorSubcoreMesh(core_axis_name="c", subcore_axis_name="s")

    @pl.kernel(
        out_type=jax.ShapeDtypeStruct((N, 128), jnp.float32),
        mesh=mesh,
        scratch_types=(
            [pltpu.VMEM((per_w,), jnp.int32)]
            + [pltpu.VMEM((WINDOW, D_MODEL), jnp.float32) for _ in range(NBUF)]
            + [pltpu.SemaphoreType.DMA for _ in range(2 * NBUF + 1)]
        ),
        compiler_params=pltpu.CompilerParams(use_tc_tiling_on_sc=False),
    )
    def emb_kernel(tbl_hbm, i_hbm, o_hbm, idx_v,
                   b0, b1, b2, b3, b4, b5, b6, b7,
                   g0, g1, g2, g3, g4, g5, g6, g7,
                   o0, o1, o2, o3, o4, o5, o6, o7, isem):
        bufs = (b0, b1, b2, b3, b4, b5, b6, b7)
        gsems = (g0, g1, g2, g3, g4, g5, g6, g7)
        osems = (o0, o1, o2, o3, o4, o5, o6, o7)
        wid = lax.axis_index("s") * 2 + lax.axis_index("c")
        base = wid * per_w

        pltpu.async_copy(i_hbm.at[0, pl.ds(base, per_w)], idx_v, isem).wait()

        def idx_slice(k):
            return idx_v.at[pl.ds(k * WINDOW, WINDOW)]

        def out_slice(k):
            return o_hbm.at[pl.ds(base + k * WINDOW, WINDOW),
                            pl.ds(0, D_MODEL)]

        def fire_gather(k, b):
            pltpu.async_copy(tbl_hbm.at[idx_slice(k)], bufs[b], gsems[b])

        for k in range(LOOKAHEAD):
            fire_gather(k, k % NBUF)

        @pl.loop(0, n_win, step=NBUF)
        def _(k0):
            for b in range(NBUF):
                k = k0 + b
                pltpu.make_async_copy(
                    tbl_hbm.at[idx_slice(k)], bufs[b], gsems[b]).wait()
                pltpu.async_copy(bufs[b], out_slice(k), osems[b])

                bn = (b + LOOKAHEAD) % NBUF
                kn = k + LOOKAHEAD

                @pl.when(k >= NBUF - LOOKAHEAD)
                def _():
                    pltpu.make_async_copy(
                        bufs[bn], out_slice(kn - NBUF), osems[bn]).wait()

                @pl.when(kn < n_win)
                def _():
                    fire_gather(kn, bn)

        for k in range(n_win - (NBUF - LOOKAHEAD), n_win):
            b = k % NBUF
            pltpu.make_async_copy(bufs[b], out_slice(k), osems[b]).wait()

    out = emb_kernel(tbl2, idx)
    return out[:, :D_MODEL].reshape(B, S, D_MODEL)
